# baseline (speedup 1.0000x reference)
import jax
import jax.numpy as jnp
from jax import lax
from jax.experimental import pallas as pl
from jax.experimental.pallas import tpu as pltpu
from jax.experimental.pallas import tpu_sc as plsc

N = 10000
E = 320000
D = 128

NC = 2
NS = 16
DH = D // NC
NH = N // 2

EPAD = 327680
E_PER_T = EPAD // NS
CHUNK = 64
N_CHUNKS = E_PER_T // CHUNK
NPAD = 10240
NPAD2 = NPAD // 2
ROWS_PER_TILE = NPAD2 // NS
LANES = 16
G16 = CHUNK // LANES

NBUF = 2
NIDX = 8


def _sc_scatter(state0_hbm, state1_hbm, w_hbm, row_hbm, col_hbm,
                zeros_hbm, zerosd_hbm,
                out_hbm, deg_hbm,
                sstate, accum, degsh,
                irs, ics, wbs, irps, icps, ras,
                cexp, aexp, bexp,
                sem_is, sem_gs):
    c = lax.axis_index("c")
    s = lax.axis_index("s")
    base0 = s * E_PER_T

    @pl.when(s < 5)
    def _():
        ssl = pl.ds(s * 1000, 1000)

        @pl.when(c == 0)
        def _():
            pltpu.sync_copy(state0_hbm.at[ssl], sstate.at[ssl])

        @pl.when(c == 1)
        def _():
            pltpu.sync_copy(state1_hbm.at[ssl], sstate.at[ssl])

    zsl = pl.ds(s * ROWS_PER_TILE, ROWS_PER_TILE)
    pltpu.sync_copy(zeros_hbm.at[zsl], accum.at[zsl])

    @pl.when(s == 0)
    def _():
        pltpu.sync_copy(zerosd_hbm, degsh)
    plsc.subcore_barrier()

    def issue_idx(g, j):
        base = base0 + g * CHUNK
        pltpu.async_copy(row_hbm.at[pl.ds(base, CHUNK)], irs[j], sem_is[j])
        pltpu.async_copy(col_hbm.at[pl.ds(base, CHUNK)], ics[j], sem_is[j])
        pltpu.async_copy(w_hbm.at[pl.ds(base, CHUNK)], wbs[j], sem_is[j])

    def wait_idx(j):
        z = pl.ds(0, CHUNK)
        pltpu.make_async_copy(row_hbm.at[z], irs[j], sem_is[j]).wait()
        pltpu.make_async_copy(col_hbm.at[z], ics[j], sem_is[j]).wait()
        pltpu.make_async_copy(w_hbm.at[z], wbs[j], sem_is[j]).wait()

    def halve_cols(j):
        def body(k, _):
            kl = pl.ds(k * LANES, LANES)
            icps[j][kl] = jax.lax.shift_right_logical(ics[j][kl], 1)
            return _
        lax.fori_loop(0, G16, body, None)

    for j in range(NIDX):
        issue_idx(j, j)
    for b in range(NBUF):
        wait_idx(b)
        halve_cols(b)
        pltpu.async_copy(sstate.at[icps[b]], ras[b], sem_gs[b])

    def group(i2, _):
        h0 = i2 * NIDX
        for v in range(NIDX):
            h = h0 + v
            jr = v % NBUF
            ji = v
            pv = (v + NBUF - 1) % NIDX
            pr = pv % NBUF
            pre = h + NBUF - 1

            @pl.when(jnp.logical_and(pre >= NBUF, pre < N_CHUNKS))
            def _():
                wait_idx(pv)
                halve_cols(pv)
                pltpu.async_copy(sstate.at[icps[pv]], ras[pr], sem_gs[pr])

            @pl.when(c == 0)
            def _():
                pltpu.sync_copy(wbs[ji], degsh.at[irs[ji]], add=True)

            def coeffs(k, _):
                kl = pl.ds(k * LANES, LANES)
                irv = irs[ji][kl]
                icv = ics[ji][kl]
                wv = wbs[ji][kl]
                irps[ji][kl] = jax.lax.shift_right_logical(irv, 1)
                pcv = jnp.bitwise_and(icv, 1).astype(jnp.float32)
                prv = jnp.bitwise_and(irv, 1).astype(jnp.float32)
                av = wv * (1.0 - prv)
                bv = wv * prv
                for e16 in range(LANES):
                    cexp[k * LANES + e16, :] = jnp.full((LANES,), pcv[e16], jnp.float32)
                    aexp[k * LANES + e16, :] = jnp.full((LANES,), av[e16], jnp.float32)
                    bexp[k * LANES + e16, :] = jnp.full((LANES,), bv[e16], jnp.float32)
                return _
            lax.fori_loop(0, G16, coeffs, None)

            pltpu.make_async_copy(sstate.at[icps[ji]], ras[jr], sem_gs[jr]).wait()

            def edge(e, _):
                cm = cexp[e, :]
                am = aexp[e, :]
                bm = bexp[e, :]
                for j in range(DH // LANES):
                    lo = pl.ds(j * LANES, LANES)
                    hi = pl.ds(DH + j * LANES, LANES)
                    glo = ras[jr][e, lo]
                    ghi = ras[jr][e, hi]
                    g = glo + cm * (ghi - glo)
                    ras[jr][e, lo] = am * g
                    ras[jr][e, hi] = bm * g
                return _
            lax.fori_loop(0, CHUNK, edge, None)
            pltpu.sync_copy(ras[jr], accum.at[irps[ji]], add=True)

            @pl.when(h + NIDX < N_CHUNKS)
            def _():
                issue_idx(h + NIDX, ji)
        return _
    lax.fori_loop(0, N_CHUNKS // NIDX, group, None)

    plsc.subcore_barrier()
    sl = pl.ds(s * ROWS_PER_TILE, ROWS_PER_TILE)
    pltpu.sync_copy(accum.at[sl], out_hbm.at[c, sl])

    @pl.when(jnp.logical_and(s == 0, c == 0))
    def _():
        pltpu.sync_copy(degsh, deg_hbm)


def _tc_combine(p_ref, deg_ref, state_ref, o_ref):
    deg = deg_ref[...]
    o_ref[...] = jnp.concatenate(
        [p_ref[0] - deg[:, None] * state_ref[:, :DH],
         p_ref[1] - deg[:, None] * state_ref[:, DH:]], axis=1)


@jax.jit
def kernel(state_variable, weights, edge_index):
    npad_e = EPAD - E
    row = jnp.concatenate([edge_index[0], jnp.full((npad_e,), N, jnp.int32)])
    col = jnp.concatenate([edge_index[1], jnp.zeros((npad_e,), jnp.int32)])
    weights = jnp.concatenate([weights, jnp.zeros((npad_e,), jnp.float32)])
    state0 = state_variable[:, :DH].reshape(NH, D)
    state1 = state_variable[:, DH:].reshape(NH, D)
    mesh = plsc.VectorSubcoreMesh(core_axis_name="c", subcore_axis_name="s")
    partial, deg = pl.kernel(
        _sc_scatter,
        mesh=mesh,
        compiler_params=pltpu.CompilerParams(needs_layout_passes=False),
        out_type=(
            jax.ShapeDtypeStruct((NC, NPAD2, D), jnp.float32),
            jax.ShapeDtypeStruct((NPAD,), jnp.float32),
        ),
        scratch_types=[
            pltpu.VMEM_SHARED((NH, D), jnp.float32),
            pltpu.VMEM_SHARED((NPAD2, D), jnp.float32),
            pltpu.VMEM_SHARED((NPAD,), jnp.float32),
            [pltpu.VMEM((CHUNK,), jnp.int32) for _ in range(NIDX)],
            [pltpu.VMEM((CHUNK,), jnp.int32) for _ in range(NIDX)],
            [pltpu.VMEM((CHUNK,), jnp.float32) for _ in range(NIDX)],
            [pltpu.VMEM((CHUNK,), jnp.int32) for _ in range(NIDX)],
            [pltpu.VMEM((CHUNK,), jnp.int32) for _ in range(NIDX)],
            [pltpu.VMEM((CHUNK, D), jnp.float32) for _ in range(NBUF)],
            pltpu.VMEM((CHUNK, LANES), jnp.float32),
            pltpu.VMEM((CHUNK, LANES), jnp.float32),
            pltpu.VMEM((CHUNK, LANES), jnp.float32),
            [pltpu.SemaphoreType.DMA for _ in range(NIDX)],
            [pltpu.SemaphoreType.DMA for _ in range(NBUF)],
        ],
    )(state0, state1, weights, row, col,
      jnp.zeros((NPAD2, D), jnp.float32), jnp.zeros((NPAD,), jnp.float32))

    partial = partial.reshape(NC, NPAD, DH)
    nblk = 10
    blk = NPAD // nblk
    return pl.pallas_call(
        _tc_combine,
        grid=(nblk,),
        in_specs=[
            pl.BlockSpec((NC, blk, DH), lambda i: (0, i, 0)),
            pl.BlockSpec((blk,), lambda i: (i,)),
            pl.BlockSpec((blk, D), lambda i: (i, 0)),
        ],
        out_specs=pl.BlockSpec((blk, D), lambda i: (i, 0)),
        out_shape=jax.ShapeDtypeStruct((N, D), jnp.float32),
    )(partial, deg, state_variable)
